# baseline (speedup 1.0000x reference)
import functools
import jax
import jax.numpy as jnp
from jax import lax
from jax.experimental import pallas as pl
from jax.experimental.pallas import tpu as pltpu
from jax.experimental.pallas import tpu_sc as plsc

N = 10000
E = 320000
F_IN = 128
HID = 64
HEADS = 4
NEG = 0.2

NSUB = 16
NCORE = 2
CH = 80
CPB = 12
E2 = 322560
EPT = E2 // NSUB
NCH = EPT // CH
G3 = NCH // 3
IB = CPB * CH
RPS = 632
NP = NSUB * RPS

f32 = jnp.float32
i32 = jnp.int32


def _leaky(v):
    return jnp.where(v >= 0, v, NEG * v)


BR = 2000
NG = N // BR


def _stage_a(x, w1t, acat1, wint, b_in):
    def body(x_ref, w1t_ref, acat1_ref, wint_ref, bin_ref,
             h1p_ref, sc1_ref, m1_ref, xw_ref, msc):
        i = pl.program_id(0)
        X = x_ref[:]
        w1t = w1t_ref[:]
        zpad = jnp.zeros((BR, 2), f32)
        h1p_ref[0] = jnp.concatenate(
            [jnp.dot(X, w1t[:, :HID * 2], preferred_element_type=f32), zpad],
            axis=1)
        h1p_ref[1] = jnp.concatenate(
            [jnp.dot(X, w1t[:, HID * 2:], preferred_element_type=f32), zpad],
            axis=1)
        A = acat1_ref[:]
        aat = jnp.dot(X, A.T, preferred_element_type=f32)
        sc1_ref[:] = jnp.concatenate([aat, jnp.zeros((BR, 8), f32)], axis=1)
        bmx = jnp.max(aat, axis=0, keepdims=True)

        @pl.when(i == 0)
        def _():
            msc[:] = jnp.full((1, 8), -jnp.inf, f32)

        msc[:] = jnp.maximum(msc[:], bmx)
        mx = msc[:][0]
        m1 = mx[:HEADS] + mx[HEADS:]
        m1_ref[:] = jnp.concatenate(
            [jnp.broadcast_to(m1[:, None], (HEADS, 16)),
             jnp.zeros((8 - HEADS, 16), f32)], axis=0)
        xw_ref[:] = jnp.dot(X, wint_ref[:], preferred_element_type=f32) + bin_ref[:]

    full = lambda *shape: pl.BlockSpec(shape, lambda i: (0,) * len(shape))
    return pl.pallas_call(
        body,
        grid=(NG,),
        in_specs=[
            pl.BlockSpec((BR, F_IN), lambda i: (i, 0)),
            full(F_IN, 4 * HID),
            full(8, F_IN),
            full(F_IN, HID),
            full(1, HID),
        ],
        out_specs=[
            pl.BlockSpec((2, BR, 2 * HID + 2), lambda i: (0, i, 0)),
            pl.BlockSpec((BR, 16), lambda i: (i, 0)),
            pl.BlockSpec((8, 16), lambda i: (0, 0)),
            pl.BlockSpec((BR, HID), lambda i: (i, 0)),
        ],
        scratch_shapes=[pltpu.VMEM((1, 8), f32)],
        out_shape=[
            jax.ShapeDtypeStruct((2, N, 2 * HID + 2), f32),
            jax.ShapeDtypeStruct((N, 16), f32),
            jax.ShapeDtypeStruct((8, 16), f32),
            jax.ShapeDtypeStruct((N, HID), f32),
        ],
    )(x, w1t, acat1, wint, b_in)


@functools.lru_cache(maxsize=None)
def _make_sc_pass(n_cols, n_heads, n_rows_aa):
    cph = n_cols // n_heads
    nct = n_cols + n_heads
    mesh = plsc.VectorSubcoreMesh(core_axis_name="c", subcore_axis_name="s")

    def body(h_hbm, gsrc_hbm, dst_hbm, sc_hbm, m_hbm, zc_hbm,
             acc_out,
             acc, m_v, gsrcb_v, dstb_v,
             rows0, rows1, rows2, asg0, asg1, asg2, adg0, adg1, adg2,
             sd0, sd1, sd2, wlin_v,
             gsem0, gsem1, gsem2, ssem0, ssem1, ssem2):
        rows = (rows0, rows1, rows2)
        asg = (asg0, asg1, asg2)
        adg = (adg0, adg1, adg2)
        sdst = (sd0, sd1, sd2)
        gsem = (gsem0, gsem1, gsem2)
        ssem = (ssem0, ssem1, ssem2)
        c = lax.axis_index("c")
        s = lax.axis_index("s")
        ebase = s * EPT
        lane = jax.lax.iota(i32, 16)

        pltpu.sync_copy(m_hbm, m_v)
        pltpu.sync_copy(zc_hbm.at[pl.ds(s * RPS, RPS)],
                        acc.at[pl.ds(s * RPS, RPS)])
        plsc.subcore_barrier()
        if n_rows_aa == 8:
            mhs = [plsc.load_gather(m_v, [jnp.full((16,), 16, i32) * (2 * c + hl)])
                   for hl in range(n_heads)]
        else:
            mhs = [plsc.load_gather(m_v, [jnp.zeros((16,), i32)])]

        def refill(blk):
            pltpu.sync_copy(
                gsrc_hbm.at[pl.ds(c * E2 + ebase + blk * IB, IB)], gsrcb_v)
            pltpu.sync_copy(dst_hbm.at[pl.ds(ebase + blk * IB, IB)], dstb_v)

        def issue_gathers(local, b):
            off = local * CH
            pltpu.async_copy(h_hbm.at[gsrcb_v.at[pl.ds(off, CH)]],
                             rows[b], gsem[b])
            pltpu.async_copy(sc_hbm.at[gsrcb_v.at[pl.ds(off, CH)]],
                             asg[b], gsem[b])
            pltpu.async_copy(sc_hbm.at[dstb_v.at[pl.ds(off, CH)]],
                             adg[b], gsem[b])

        def wait_gathers(b):
            z = pl.ds(0, CH)
            pltpu.make_async_copy(h_hbm.at[gsrcb_v.at[z]], rows[b],
                                  gsem[b]).wait()
            pltpu.make_async_copy(sc_hbm.at[gsrcb_v.at[z]], asg[b],
                                  gsem[b]).wait()
            pltpu.make_async_copy(sc_hbm.at[dstb_v.at[z]], adg[b],
                                  gsem[b]).wait()

        def wait_scatter(b):
            pltpu.make_async_copy(rows[b], acc.at[sdst[b]], ssem[b]).wait()

        def copy_sdst(local, b):
            off = local * CH
            for j in range(CH // 16):
                sdst[b][pl.ds(j * 16, 16)] = dstb_v[pl.ds(off + j * 16, 16)]

        def compute_and_scatter(b):
            for j in range(CH // 16):
                lanes = lane + j * 16
                for hl in range(n_heads):
                    if n_rows_aa == 8:
                        scol = 2 * c + hl
                        dcol = 4 + 2 * c + hl
                    else:
                        scol = 0
                        dcol = 1
                    mh = mhs[hl]
                    a_s = plsc.load_gather(
                        asg[b], [lanes, jnp.full((16,), scol, i32)])
                    a_d = plsc.load_gather(
                        adg[b], [lanes, jnp.full((16,), dcol, i32)])
                    e = a_s + a_d
                    e = jnp.where(e >= 0, e, NEG * e)
                    w = jnp.exp(e - mh)
                    wlin_v[pl.ds(hl * CH + j * 16, 16)] = w

            rb = rows[b]

            def scale_one(ei, carry2):
                wmix = jnp.zeros((16,), f32)
                for hl in range(n_heads):
                    wv = plsc.load_gather(
                        wlin_v, [jnp.full((16,), hl * CH, i32) + ei])
                    for q in range(cph // 16):
                        sl = pl.ds(hl * cph + q * 16, 16)
                        rb[ei, sl] = rb[ei, sl] * wv
                    wmix = jnp.where(lane == hl, wv, wmix)
                plsc.store_scatter(
                    rb, [jnp.full((16,), ei, i32), n_cols + lane], wmix,
                    mask=lane < n_heads)
                return carry2

            lax.fori_loop(0, CH, scale_one, 0, unroll=4)
            pltpu.async_copy(rb, acc.at[sdst[b]], ssem[b], add=True)

        refill(0)
        issue_gathers(0, 0)
        gpb = CPB // 3

        def group(g, carry):
            lg = lax.rem(g, gpb)
            for b in range(3):
                bn = (b + 1) % 3
                copy_sdst(3 * lg + b, b)
                wait_gathers(b)
                if b < 2:
                    @pl.when(g >= 1)
                    def _():
                        wait_scatter(bn)
                    issue_gathers(3 * lg + b + 1, bn)
                else:
                    wait_scatter(bn)

                    @pl.when(g < G3 - 1)
                    def _():
                        @pl.when(lg == gpb - 1)
                        def _():
                            refill((g + 1) // gpb)
                        issue_gathers(3 * lax.rem(g + 1, gpb), bn)
                compute_and_scatter(b)
            return carry

        lax.fori_loop(0, G3, group, 0)
        wait_scatter(1)
        wait_scatter(2)
        plsc.subcore_barrier()
        pltpu.sync_copy(acc.at[pl.ds(s * RPS, RPS)],
                        acc_out.at[pl.ds(c * NP + s * RPS, RPS)])

    return functools.partial(
        pl.kernel,
        mesh=mesh,
        compiler_params=pltpu.CompilerParams(
            needs_layout_passes=False, use_tc_tiling_on_sc=False),
        out_type=[
            jax.ShapeDtypeStruct((NCORE * NP, nct), f32),
        ],
        scratch_types=[
            pltpu.VMEM_SHARED((NP, nct), f32),
            pltpu.VMEM((128,), f32),
            pltpu.VMEM((IB,), i32),
            pltpu.VMEM((IB,), i32),
            pltpu.VMEM((CH, nct), f32),
            pltpu.VMEM((CH, nct), f32),
            pltpu.VMEM((CH, nct), f32),
            pltpu.VMEM((CH, 16), f32),
            pltpu.VMEM((CH, 16), f32),
            pltpu.VMEM((CH, 16), f32),
            pltpu.VMEM((CH, 16), f32),
            pltpu.VMEM((CH, 16), f32),
            pltpu.VMEM((CH, 16), f32),
            pltpu.VMEM((CH,), i32),
            pltpu.VMEM((CH,), i32),
            pltpu.VMEM((CH,), i32),
            pltpu.VMEM((n_heads * CH,), f32),
            pltpu.SemaphoreType.DMA,
            pltpu.SemaphoreType.DMA,
            pltpu.SemaphoreType.DMA,
            pltpu.SemaphoreType.DMA,
            pltpu.SemaphoreType.DMA,
            pltpu.SemaphoreType.DMA,
        ],
    )(body)


def _stage_b(acc1, h1p, sc1, m1, w2t, b1, acat2, kmat):
    def body(acc1_ref, h1p_ref, sc1_ref, m1_ref, w2t_ref,
             b1_ref, acat2_ref, kmat_ref,
             h2f_ref, sc2_ref, m2_ref, msc):
        i = pl.program_id(0)
        C2 = 2 * HID
        a0 = acc1_ref[0]
        a1 = acc1_ref[1]
        out1 = jnp.concatenate([a0[:, :C2], a1[:, :C2]], axis=1)
        h1 = jnp.concatenate([h1p_ref[0][:, :C2], h1p_ref[1][:, :C2]], axis=1)
        den4 = jnp.concatenate([a0[:, C2:C2 + 2], a1[:, C2:C2 + 2]],
                               axis=1)
        aat = sc1_ref[:]
        m1v = m1_ref[:][:HEADS, 0]
        es = _leaky(aat[:, :HEADS] + aat[:, HEADS:2 * HEADS])
        wself = jnp.exp(es - m1v[None, :])
        K = kmat_ref[:]
        wb = jnp.dot(wself, K, preferred_element_type=f32)
        db = jnp.dot(den4 + wself, K, preferred_element_type=f32) + 1e-16
        g1 = _leaky((out1 + wb * h1) / db + b1_ref[:])
        h2 = jnp.dot(g1, w2t_ref[:], preferred_element_type=f32)
        zpad = jnp.zeros((BR, 1), f32)
        h2f_ref[0] = jnp.concatenate([h2[:, :HID // 2], zpad], axis=1)
        h2f_ref[1] = jnp.concatenate([h2[:, HID // 2:], zpad], axis=1)
        A2 = acat2_ref[:]
        av = jnp.dot(h2, A2.T, preferred_element_type=f32)
        sc2_ref[:] = jnp.concatenate([av, jnp.zeros((BR, 14), f32)], axis=1)
        bmx = jnp.max(av, axis=0, keepdims=True)

        @pl.when(i == 0)
        def _():
            msc[:] = jnp.full((1, 2), -jnp.inf, f32)

        msc[:] = jnp.maximum(msc[:], bmx)
        mx = msc[:][0]
        m2_ref[:] = jnp.concatenate(
            [jnp.broadcast_to(mx[:1] + mx[1:], (1, 16)),
             jnp.zeros((7, 16), f32)], axis=0)

    full = lambda *shape: pl.BlockSpec(shape, lambda i: (0,) * len(shape))
    return pl.pallas_call(
        body,
        grid=(NG,),
        in_specs=[
            pl.BlockSpec((2, BR, 2 * HID + 2), lambda i: (0, i, 0)),
            pl.BlockSpec((2, BR, 2 * HID + 2), lambda i: (0, i, 0)),
            pl.BlockSpec((BR, 16), lambda i: (i, 0)),
            full(8, 16),
            full(4 * HID, HID),
            full(1, 4 * HID),
            full(2, HID),
            full(HEADS, 4 * HID),
        ],
        out_specs=[
            pl.BlockSpec((2, BR, HID // 2 + 1), lambda i: (0, i, 0)),
            pl.BlockSpec((BR, 16), lambda i: (i, 0)),
            pl.BlockSpec((8, 16), lambda i: (0, 0)),
        ],
        scratch_shapes=[pltpu.VMEM((1, 2), f32)],
        out_shape=[
            jax.ShapeDtypeStruct((2, N, HID // 2 + 1), f32),
            jax.ShapeDtypeStruct((N, 16), f32),
            jax.ShapeDtypeStruct((8, 16), f32),
        ],
    )(acc1, h1p, sc1, m1, w2t, b1, acat2, kmat)


def _stage_c(acc2, h2f, sc2, m2, xw, b2, gamma, beta,
             wiht, b_ih, b_hh, wfct, b_fc):
    def body(acc2_ref, h2f_ref, sc2_ref, m2_ref, xw_ref,
             b2_ref, gamma_ref, beta_ref, wiht_ref, bih_ref, bhh_ref,
             wfct_ref, bfc_ref, out_ref):
        CC = HID // 2
        b0 = acc2_ref[0]
        b1v = acc2_ref[1]
        out2 = jnp.concatenate([b0[:, :CC], b1v[:, :CC]], axis=1)
        h2 = jnp.concatenate([h2f_ref[0][:, :CC], h2f_ref[1][:, :CC]], axis=1)
        den = b0[:, CC:CC + 1]
        at = sc2_ref[:]
        wself = jnp.exp(_leaky(at[:, :1] + at[:, 1:2]) - m2_ref[0, 0])
        g2 = (out2 + wself * h2) / (den + wself + 1e-16)
        h = _leaky(g2 + b2_ref[:])
        mu = jnp.mean(h, axis=1, keepdims=True)
        d = h - mu
        var = jnp.mean(d * d, axis=1, keepdims=True)
        h = d * jax.lax.rsqrt(var + 1e-5) * gamma_ref[:] + beta_ref[:]
        h = h + xw_ref[:]
        gi = jnp.dot(h, wiht_ref[:], preferred_element_type=f32) + bih_ref[:]
        bhh = bhh_ref[:]
        r = jax.nn.sigmoid(gi[:, :HID] + bhh[:, :HID])
        z = jax.nn.sigmoid(gi[:, HID:2 * HID] + bhh[:, HID:2 * HID])
        nc = jnp.tanh(gi[:, 2 * HID:] + r * bhh[:, 2 * HID:])
        hout = (1.0 - z) * nc
        out_ref[:] = jnp.dot(hout, wfct_ref[:],
                             preferred_element_type=f32) + bfc_ref[:]

    full = lambda *shape: pl.BlockSpec(shape, lambda i: (0,) * len(shape))
    return pl.pallas_call(
        body,
        grid=(NG,),
        in_specs=[
            pl.BlockSpec((2, BR, HID // 2 + 1), lambda i: (0, i, 0)),
            pl.BlockSpec((2, BR, HID // 2 + 1), lambda i: (0, i, 0)),
            pl.BlockSpec((BR, 16), lambda i: (i, 0)),
            full(8, 16),
            pl.BlockSpec((BR, HID), lambda i: (i, 0)),
            full(1, HID),
            full(1, HID),
            full(1, HID),
            full(HID, 3 * HID),
            full(1, 3 * HID),
            full(1, 3 * HID),
            full(HID, 8),
            full(1, 8),
        ],
        out_specs=pl.BlockSpec((BR, 8), lambda i: (i, 0)),
        out_shape=jax.ShapeDtypeStruct((N, 8), f32),
    )(acc2, h2f, sc2, m2, xw, b2, gamma, beta,
      wiht, b_ih, b_hh, wfct, b_fc)


def _sc_pass1(*args):
    return _make_sc_pass(2 * HID, 2, 8)(*args)


def _sc_pass2(*args):
    return _make_sc_pass(HID // 2, 1, 2)(*args)


@jax.jit
def kernel(x, edge_index, W_in, b_in, W1, a_src1, a_dst1, b1, W2, a_src2,
           a_dst2, b2, gamma, beta, W_ih, W_hh, b_ih, b_hh, W_fc, b_fc):
    w1t = W1.T
    W1h = W1.reshape(HEADS, HID, F_IN)
    as_rows = jnp.einsum('hcf,hc->hf', W1h, a_src1)
    ad_rows = jnp.einsum('hcf,hc->hf', W1h, a_dst1)
    acat1 = jnp.concatenate([as_rows, ad_rows], axis=0)
    wint = W_in.T
    w2t = W2.T
    acat2 = jnp.concatenate([a_src2, a_dst2], axis=0)
    kmat = jnp.kron(jnp.eye(HEADS, dtype=f32), jnp.ones((1, HID), f32))
    wiht = W_ih.T
    wfct = jnp.concatenate(
        [W_fc.T, jnp.zeros((HID, 5), f32)], axis=1)
    bfc = jnp.concatenate([b_fc, jnp.zeros((5,), f32)])[None, :]

    pade = E2 - E
    src_p = jnp.concatenate([edge_index[0], jnp.zeros((pade,), i32)])
    dst_p = jnp.concatenate([edge_index[1], jnp.full((pade,), N, i32)])
    gsrc_all = jnp.concatenate([src_p, src_p + N])
    zc1 = jnp.zeros((NP, 2 * HID + 2), f32)
    zc2 = jnp.zeros((NP, HID // 2 + 1), f32)

    h1p, sc1, m1, xw = _stage_a(x, w1t, acat1, wint, b_in[None, :])
    sc1x = jnp.concatenate([sc1, sc1], axis=0)

    acc1 = _sc_pass1(h1p.reshape(NCORE * N, 2 * HID + 2), gsrc_all, dst_p,
                     sc1x, m1.reshape(-1), zc1)
    acc1 = acc1[0] if isinstance(acc1, (list, tuple)) else acc1

    acc1c = jnp.stack([acc1[:N], acc1[NP:NP + N]])
    h2f, sc2, m2 = _stage_b(
        acc1c, h1p, sc1, m1, w2t, b1[None, :], acat2, kmat)
    sc2x = jnp.concatenate([sc2, sc2], axis=0)

    acc2 = _sc_pass2(h2f.reshape(NCORE * N, HID // 2 + 1), gsrc_all, dst_p,
                     sc2x, m2.reshape(-1), zc2)
    acc2 = acc2[0] if isinstance(acc2, (list, tuple)) else acc2

    acc2c = jnp.stack([acc2[:N], acc2[NP:NP + N]])
    out = _stage_c(acc2c, h2f, sc2, m2, xw,
                   b2[None, :], gamma[None, :], beta[None, :],
                   wiht, b_ih[None, :], b_hh[None, :], wfct, bfc)
    return out[:, :3]


# speedup vs baseline: 1.2733x; 1.2733x over previous
import functools
import jax
import jax.numpy as jnp
from jax import lax
from jax.experimental import pallas as pl
from jax.experimental.pallas import tpu as pltpu
from jax.experimental.pallas import tpu_sc as plsc

N = 10000
E = 320000
F_IN = 128
HID = 64
HEADS = 4
NEG = 0.2

NSUB = 16
NCORE = 2
CH = 80
CPB = 12
E2 = 322560
EPT = E2 // NSUB
NCH = EPT // CH
G3 = NCH // 3
IB = CPB * CH
RPS = 632
NP = NSUB * RPS

f32 = jnp.float32
i32 = jnp.int32


def _leaky(v):
    return jnp.where(v >= 0, v, NEG * v)


BR = 2000
NG = N // BR


def _stage_a(x, w1t, acat1, wint, b_in):
    def body(x_ref, w1t_ref, acat1_ref, wint_ref, bin_ref,
             h1p_ref, sc1_ref, m1_ref, xw_ref, msc):
        i = pl.program_id(0)
        X = x_ref[:]
        w1t = w1t_ref[:]
        zpad = jnp.zeros((BR, 2), f32)
        h1p_ref[0] = jnp.concatenate(
            [jnp.dot(X, w1t[:, :HID * 2], preferred_element_type=f32), zpad],
            axis=1)
        h1p_ref[1] = jnp.concatenate(
            [jnp.dot(X, w1t[:, HID * 2:], preferred_element_type=f32), zpad],
            axis=1)
        A = acat1_ref[:]
        aat = jnp.dot(X, A.T, preferred_element_type=f32)
        sc1_ref[:] = jnp.concatenate([aat, jnp.zeros((BR, 8), f32)], axis=1)
        bmx = jnp.max(aat, axis=0, keepdims=True)

        @pl.when(i == 0)
        def _():
            msc[:] = jnp.full((1, 8), -jnp.inf, f32)

        msc[:] = jnp.maximum(msc[:], bmx)
        mx = msc[:][0]
        m1 = mx[:HEADS] + mx[HEADS:]
        m1_ref[:] = jnp.concatenate(
            [jnp.broadcast_to(m1[:, None], (HEADS, 16)),
             jnp.zeros((8 - HEADS, 16), f32)], axis=0)
        xw_ref[:] = jnp.dot(X, wint_ref[:], preferred_element_type=f32) + bin_ref[:]

    full = lambda *shape: pl.BlockSpec(shape, lambda i: (0,) * len(shape))
    return pl.pallas_call(
        body,
        grid=(NG,),
        in_specs=[
            pl.BlockSpec((BR, F_IN), lambda i: (i, 0)),
            full(F_IN, 4 * HID),
            full(8, F_IN),
            full(F_IN, HID),
            full(1, HID),
        ],
        out_specs=[
            pl.BlockSpec((2, BR, 2 * HID + 2), lambda i: (0, i, 0)),
            pl.BlockSpec((BR, 16), lambda i: (i, 0)),
            pl.BlockSpec((8, 16), lambda i: (0, 0)),
            pl.BlockSpec((BR, HID), lambda i: (i, 0)),
        ],
        scratch_shapes=[pltpu.VMEM((1, 8), f32)],
        out_shape=[
            jax.ShapeDtypeStruct((2, N, 2 * HID + 2), f32),
            jax.ShapeDtypeStruct((N, 16), f32),
            jax.ShapeDtypeStruct((8, 16), f32),
            jax.ShapeDtypeStruct((N, HID), f32),
        ],
    )(x, w1t, acat1, wint, b_in)


@functools.lru_cache(maxsize=None)
def _make_sc_pass(n_cols, n_heads, n_rows_aa):
    cph = n_cols // n_heads
    nct = n_cols + n_heads
    mesh = plsc.VectorSubcoreMesh(core_axis_name="c", subcore_axis_name="s")

    def body(h_hbm, gsrc_hbm, dst_hbm, sc_hbm, m_hbm, zc_hbm,
             acc_out,
             acc, m_v, gsrcb_v, dstb_v,
             rows0, rows1, rows2, asg0, asg1, asg2, adg0, adg1, adg2,
             sd0, sd1, sd2, wlin_v,
             gsem0, gsem1, gsem2, ssem0, ssem1, ssem2):
        rows = (rows0, rows1, rows2)
        asg = (asg0, asg1, asg2)
        adg = (adg0, adg1, adg2)
        sdst = (sd0, sd1, sd2)
        gsem = (gsem0, gsem1, gsem2)
        ssem = (ssem0, ssem1, ssem2)
        c = lax.axis_index("c")
        s = lax.axis_index("s")
        ebase = s * EPT
        lane = jax.lax.iota(i32, 16)

        pltpu.sync_copy(m_hbm, m_v)
        pltpu.sync_copy(zc_hbm.at[pl.ds(s * RPS, RPS)],
                        acc.at[pl.ds(s * RPS, RPS)])
        plsc.subcore_barrier()
        if n_rows_aa == 8:
            mhs = [plsc.load_gather(m_v, [jnp.full((16,), 16, i32) * (2 * c + hl)])
                   for hl in range(n_heads)]
        else:
            mhs = [plsc.load_gather(m_v, [jnp.zeros((16,), i32)])]

        def refill(blk):
            pltpu.sync_copy(
                gsrc_hbm.at[pl.ds(c * E2 + ebase + blk * IB, IB)], gsrcb_v)
            pltpu.sync_copy(dst_hbm.at[pl.ds(ebase + blk * IB, IB)], dstb_v)

        def issue_gathers(local, b):
            off = local * CH
            pltpu.async_copy(h_hbm.at[gsrcb_v.at[pl.ds(off, CH)]],
                             rows[b], gsem[b])
            pltpu.async_copy(sc_hbm.at[gsrcb_v.at[pl.ds(off, CH)]],
                             asg[b], gsem[b])
            pltpu.async_copy(sc_hbm.at[dstb_v.at[pl.ds(off, CH)]],
                             adg[b], gsem[b])

        def wait_gathers(b):
            z = pl.ds(0, CH)
            pltpu.make_async_copy(h_hbm.at[gsrcb_v.at[z]], rows[b],
                                  gsem[b]).wait()
            pltpu.make_async_copy(sc_hbm.at[gsrcb_v.at[z]], asg[b],
                                  gsem[b]).wait()
            pltpu.make_async_copy(sc_hbm.at[dstb_v.at[z]], adg[b],
                                  gsem[b]).wait()

        def wait_scatter(b):
            pltpu.make_async_copy(rows[b], acc.at[sdst[b]], ssem[b]).wait()

        def copy_sdst(local, b):
            off = local * CH
            for j in range(CH // 16):
                sdst[b][pl.ds(j * 16, 16)] = dstb_v[pl.ds(off + j * 16, 16)]

        def compute_and_scatter(b):
            for j in range(CH // 16):
                lanes = lane + j * 16
                for hl in range(n_heads):
                    if n_rows_aa == 8:
                        scol = 2 * c + hl
                        dcol = 4 + 2 * c + hl
                    else:
                        scol = 0
                        dcol = 1
                    mh = mhs[hl]
                    a_s = plsc.load_gather(
                        asg[b], [lanes, jnp.full((16,), scol, i32)])
                    a_d = plsc.load_gather(
                        adg[b], [lanes, jnp.full((16,), dcol, i32)])
                    e = a_s + a_d
                    e = jnp.where(e >= 0, e, NEG * e)
                    w = jnp.exp(e - mh)
                    wlin_v[pl.ds(hl * CH + j * 16, 16)] = w

            rb = rows[b]

            def scale_one(ei, carry2):
                wmix = jnp.zeros((16,), f32)
                for hl in range(n_heads):
                    wv = plsc.load_gather(
                        wlin_v, [jnp.full((16,), hl * CH, i32) + ei])
                    for q in range(cph // 16):
                        sl = pl.ds(hl * cph + q * 16, 16)
                        rb[ei, sl] = rb[ei, sl] * wv
                    wmix = jnp.where(lane == hl, wv, wmix)
                plsc.store_scatter(
                    rb, [jnp.full((16,), ei, i32), n_cols + lane], wmix,
                    mask=lane < n_heads)
                return carry2

            lax.fori_loop(0, CH, scale_one, 0, unroll=2)
            pltpu.async_copy(rb, acc.at[sdst[b]], ssem[b], add=True)

        refill(0)
        issue_gathers(0, 0)
        gpb = CPB // 3

        def group(g, carry):
            lg = lax.rem(g, gpb)
            for b in range(3):
                bn = (b + 1) % 3
                copy_sdst(3 * lg + b, b)
                wait_gathers(b)
                if b < 2:
                    @pl.when(g >= 1)
                    def _():
                        wait_scatter(bn)
                    issue_gathers(3 * lg + b + 1, bn)
                else:
                    wait_scatter(bn)

                    @pl.when(g < G3 - 1)
                    def _():
                        @pl.when(lg == gpb - 1)
                        def _():
                            refill((g + 1) // gpb)
                        issue_gathers(3 * lax.rem(g + 1, gpb), bn)
                compute_and_scatter(b)
            return carry

        lax.fori_loop(0, G3, group, 0)
        wait_scatter(1)
        wait_scatter(2)
        plsc.subcore_barrier()
        pltpu.sync_copy(acc.at[pl.ds(s * RPS, RPS)],
                        acc_out.at[pl.ds(c * NP + s * RPS, RPS)])

    return functools.partial(
        pl.kernel,
        mesh=mesh,
        compiler_params=pltpu.CompilerParams(
            needs_layout_passes=False, use_tc_tiling_on_sc=False),
        out_type=[
            jax.ShapeDtypeStruct((NCORE * NP, nct), f32),
        ],
        scratch_types=[
            pltpu.VMEM_SHARED((NP, nct), f32),
            pltpu.VMEM((128,), f32),
            pltpu.VMEM((IB,), i32),
            pltpu.VMEM((IB,), i32),
            pltpu.VMEM((CH, nct), f32),
            pltpu.VMEM((CH, nct), f32),
            pltpu.VMEM((CH, nct), f32),
            pltpu.VMEM((CH, 16), f32),
            pltpu.VMEM((CH, 16), f32),
            pltpu.VMEM((CH, 16), f32),
            pltpu.VMEM((CH, 16), f32),
            pltpu.VMEM((CH, 16), f32),
            pltpu.VMEM((CH, 16), f32),
            pltpu.VMEM((CH,), i32),
            pltpu.VMEM((CH,), i32),
            pltpu.VMEM((CH,), i32),
            pltpu.VMEM((n_heads * CH,), f32),
            pltpu.SemaphoreType.DMA,
            pltpu.SemaphoreType.DMA,
            pltpu.SemaphoreType.DMA,
            pltpu.SemaphoreType.DMA,
            pltpu.SemaphoreType.DMA,
            pltpu.SemaphoreType.DMA,
        ],
    )(body)


def _stage_b(acc1, h1p, sc1, m1, w2t, b1, acat2, kmat):
    def body(acc1_ref, h1p_ref, sc1_ref, m1_ref, w2t_ref,
             b1_ref, acat2_ref, kmat_ref,
             h2f_ref, sc2_ref, m2_ref, msc):
        i = pl.program_id(0)
        C2 = 2 * HID
        a0 = acc1_ref[0]
        a1 = acc1_ref[1]
        out1 = jnp.concatenate([a0[:, :C2], a1[:, :C2]], axis=1)
        h1 = jnp.concatenate([h1p_ref[0][:, :C2], h1p_ref[1][:, :C2]], axis=1)
        den4 = jnp.concatenate([a0[:, C2:C2 + 2], a1[:, C2:C2 + 2]],
                               axis=1)
        aat = sc1_ref[:]
        m1v = m1_ref[:][:HEADS, 0]
        es = _leaky(aat[:, :HEADS] + aat[:, HEADS:2 * HEADS])
        wself = jnp.exp(es - m1v[None, :])
        K = kmat_ref[:]
        wb = jnp.dot(wself, K, preferred_element_type=f32)
        db = jnp.dot(den4 + wself, K, preferred_element_type=f32) + 1e-16
        g1 = _leaky((out1 + wb * h1) / db + b1_ref[:])
        h2 = jnp.dot(g1, w2t_ref[:], preferred_element_type=f32)
        zpad = jnp.zeros((BR, 1), f32)
        h2f_ref[0] = jnp.concatenate([h2[:, :HID // 2], zpad], axis=1)
        h2f_ref[1] = jnp.concatenate([h2[:, HID // 2:], zpad], axis=1)
        A2 = acat2_ref[:]
        av = jnp.dot(h2, A2.T, preferred_element_type=f32)
        sc2_ref[:] = jnp.concatenate([av, jnp.zeros((BR, 14), f32)], axis=1)
        bmx = jnp.max(av, axis=0, keepdims=True)

        @pl.when(i == 0)
        def _():
            msc[:] = jnp.full((1, 2), -jnp.inf, f32)

        msc[:] = jnp.maximum(msc[:], bmx)
        mx = msc[:][0]
        m2_ref[:] = jnp.concatenate(
            [jnp.broadcast_to(mx[:1] + mx[1:], (1, 16)),
             jnp.zeros((7, 16), f32)], axis=0)

    full = lambda *shape: pl.BlockSpec(shape, lambda i: (0,) * len(shape))
    return pl.pallas_call(
        body,
        grid=(NG,),
        in_specs=[
            pl.BlockSpec((2, BR, 2 * HID + 2), lambda i: (0, i, 0)),
            pl.BlockSpec((2, BR, 2 * HID + 2), lambda i: (0, i, 0)),
            pl.BlockSpec((BR, 16), lambda i: (i, 0)),
            full(8, 16),
            full(4 * HID, HID),
            full(1, 4 * HID),
            full(2, HID),
            full(HEADS, 4 * HID),
        ],
        out_specs=[
            pl.BlockSpec((2, BR, HID // 2 + 1), lambda i: (0, i, 0)),
            pl.BlockSpec((BR, 16), lambda i: (i, 0)),
            pl.BlockSpec((8, 16), lambda i: (0, 0)),
        ],
        scratch_shapes=[pltpu.VMEM((1, 2), f32)],
        out_shape=[
            jax.ShapeDtypeStruct((2, N, HID // 2 + 1), f32),
            jax.ShapeDtypeStruct((N, 16), f32),
            jax.ShapeDtypeStruct((8, 16), f32),
        ],
    )(acc1, h1p, sc1, m1, w2t, b1, acat2, kmat)


def _stage_c(acc2, h2f, sc2, m2, xw, b2, gamma, beta,
             wiht, b_ih, b_hh, wfct, b_fc):
    def body(acc2_ref, h2f_ref, sc2_ref, m2_ref, xw_ref,
             b2_ref, gamma_ref, beta_ref, wiht_ref, bih_ref, bhh_ref,
             wfct_ref, bfc_ref, out_ref):
        CC = HID // 2
        b0 = acc2_ref[0]
        b1v = acc2_ref[1]
        out2 = jnp.concatenate([b0[:, :CC], b1v[:, :CC]], axis=1)
        h2 = jnp.concatenate([h2f_ref[0][:, :CC], h2f_ref[1][:, :CC]], axis=1)
        den = b0[:, CC:CC + 1]
        at = sc2_ref[:]
        wself = jnp.exp(_leaky(at[:, :1] + at[:, 1:2]) - m2_ref[0, 0])
        g2 = (out2 + wself * h2) / (den + wself + 1e-16)
        h = _leaky(g2 + b2_ref[:])
        mu = jnp.mean(h, axis=1, keepdims=True)
        d = h - mu
        var = jnp.mean(d * d, axis=1, keepdims=True)
        h = d * jax.lax.rsqrt(var + 1e-5) * gamma_ref[:] + beta_ref[:]
        h = h + xw_ref[:]
        gi = jnp.dot(h, wiht_ref[:], preferred_element_type=f32) + bih_ref[:]
        bhh = bhh_ref[:]
        r = jax.nn.sigmoid(gi[:, :HID] + bhh[:, :HID])
        z = jax.nn.sigmoid(gi[:, HID:2 * HID] + bhh[:, HID:2 * HID])
        nc = jnp.tanh(gi[:, 2 * HID:] + r * bhh[:, 2 * HID:])
        hout = (1.0 - z) * nc
        out_ref[:] = jnp.dot(hout, wfct_ref[:],
                             preferred_element_type=f32) + bfc_ref[:]

    full = lambda *shape: pl.BlockSpec(shape, lambda i: (0,) * len(shape))
    return pl.pallas_call(
        body,
        grid=(NG,),
        in_specs=[
            pl.BlockSpec((2, BR, HID // 2 + 1), lambda i: (0, i, 0)),
            pl.BlockSpec((2, BR, HID // 2 + 1), lambda i: (0, i, 0)),
            pl.BlockSpec((BR, 16), lambda i: (i, 0)),
            full(8, 16),
            pl.BlockSpec((BR, HID), lambda i: (i, 0)),
            full(1, HID),
            full(1, HID),
            full(1, HID),
            full(HID, 3 * HID),
            full(1, 3 * HID),
            full(1, 3 * HID),
            full(HID, 8),
            full(1, 8),
        ],
        out_specs=pl.BlockSpec((BR, 8), lambda i: (i, 0)),
        out_shape=jax.ShapeDtypeStruct((N, 8), f32),
    )(acc2, h2f, sc2, m2, xw, b2, gamma, beta,
      wiht, b_ih, b_hh, wfct, b_fc)


def _sc_pass1(*args):
    return _make_sc_pass(2 * HID, 2, 8)(*args)


def _sc_pass2(*args):
    return _make_sc_pass(HID // 2, 1, 2)(*args)


@jax.jit
def kernel(x, edge_index, W_in, b_in, W1, a_src1, a_dst1, b1, W2, a_src2,
           a_dst2, b2, gamma, beta, W_ih, W_hh, b_ih, b_hh, W_fc, b_fc):
    w1t = W1.T
    W1h = W1.reshape(HEADS, HID, F_IN)
    as_rows = jnp.einsum('hcf,hc->hf', W1h, a_src1)
    ad_rows = jnp.einsum('hcf,hc->hf', W1h, a_dst1)
    acat1 = jnp.concatenate([as_rows, ad_rows], axis=0)
    wint = W_in.T
    w2t = W2.T
    acat2 = jnp.concatenate([a_src2, a_dst2], axis=0)
    kmat = jnp.kron(jnp.eye(HEADS, dtype=f32), jnp.ones((1, HID), f32))
    wiht = W_ih.T
    wfct = jnp.concatenate(
        [W_fc.T, jnp.zeros((HID, 5), f32)], axis=1)
    bfc = jnp.concatenate([b_fc, jnp.zeros((5,), f32)])[None, :]

    pade = E2 - E
    src_p = jnp.concatenate([edge_index[0], jnp.zeros((pade,), i32)])
    dst_p = jnp.concatenate([edge_index[1], jnp.full((pade,), N, i32)])
    gsrc_all = jnp.concatenate([src_p, src_p + N])
    zc1 = jnp.zeros((NP, 2 * HID + 2), f32)
    zc2 = jnp.zeros((NP, HID // 2 + 1), f32)

    h1p, sc1, m1, xw = _stage_a(x, w1t, acat1, wint, b_in[None, :])
    sc1x = jnp.concatenate([sc1, sc1], axis=0)

    acc1 = _sc_pass1(h1p.reshape(NCORE * N, 2 * HID + 2), gsrc_all, dst_p,
                     sc1x, m1.reshape(-1), zc1)
    acc1 = acc1[0] if isinstance(acc1, (list, tuple)) else acc1

    acc1c = jnp.stack([acc1[:N], acc1[NP:NP + N]])
    h2f, sc2, m2 = _stage_b(
        acc1c, h1p, sc1, m1, w2t, b1[None, :], acat2, kmat)
    sc2x = jnp.concatenate([sc2, sc2], axis=0)

    acc2 = _sc_pass2(h2f.reshape(NCORE * N, HID // 2 + 1), gsrc_all, dst_p,
                     sc2x, m2.reshape(-1), zc2)
    acc2 = acc2[0] if isinstance(acc2, (list, tuple)) else acc2

    acc2c = jnp.stack([acc2[:N], acc2[NP:NP + N]])
    out = _stage_c(acc2c, h2f, sc2, m2, xw,
                   b2[None, :], gamma[None, :], beta[None, :],
                   wiht, b_ih[None, :], b_hh[None, :], wfct, bfc)
    return out[:, :3]


# speedup vs baseline: 1.3034x; 1.0237x over previous
import functools
import jax
import jax.numpy as jnp
from jax import lax
from jax.experimental import pallas as pl
from jax.experimental.pallas import tpu as pltpu
from jax.experimental.pallas import tpu_sc as plsc

N = 10000
E = 320000
F_IN = 128
HID = 64
HEADS = 4
NEG = 0.2

NSUB = 16
NCORE = 2
CH = 80
CPB = 12
E2 = 322560
EPT = E2 // NSUB
NCH = EPT // CH
G3 = NCH // 3
IB = CPB * CH
RPS = 632
NP = NSUB * RPS

f32 = jnp.float32
i32 = jnp.int32


def _leaky(v):
    return jnp.where(v >= 0, v, NEG * v)


BR = 2000
NG = N // BR


def _stage_a(x, w1t, acat1, wint, b_in):
    def body(x_ref, w1t_ref, acat1_ref, wint_ref, bin_ref,
             h1p_ref, sc1_ref, m1_ref, xw_ref, msc):
        i = pl.program_id(0)
        X = x_ref[:]
        w1t = w1t_ref[:]
        zpad = jnp.zeros((BR, 2), f32)
        h1p_ref[0] = jnp.concatenate(
            [jnp.dot(X, w1t[:, :HID * 2], preferred_element_type=f32), zpad],
            axis=1)
        h1p_ref[1] = jnp.concatenate(
            [jnp.dot(X, w1t[:, HID * 2:], preferred_element_type=f32), zpad],
            axis=1)
        A = acat1_ref[:]
        aat = jnp.dot(X, A.T, preferred_element_type=f32)
        sc1_ref[:] = jnp.concatenate([aat, jnp.zeros((BR, 8), f32)], axis=1)
        bmx = jnp.max(aat, axis=0, keepdims=True)

        @pl.when(i == 0)
        def _():
            msc[:] = jnp.full((1, 8), -jnp.inf, f32)

        msc[:] = jnp.maximum(msc[:], bmx)
        mx = msc[:][0]
        m1 = mx[:HEADS] + mx[HEADS:]
        m1_ref[:] = jnp.concatenate(
            [jnp.broadcast_to(m1[:, None], (HEADS, 16)),
             jnp.zeros((8 - HEADS, 16), f32)], axis=0)
        xw_ref[:] = jnp.dot(X, wint_ref[:], preferred_element_type=f32) + bin_ref[:]

    full = lambda *shape: pl.BlockSpec(shape, lambda i: (0,) * len(shape))
    return pl.pallas_call(
        body,
        grid=(NG,),
        in_specs=[
            pl.BlockSpec((BR, F_IN), lambda i: (i, 0)),
            full(F_IN, 4 * HID),
            full(8, F_IN),
            full(F_IN, HID),
            full(1, HID),
        ],
        out_specs=[
            pl.BlockSpec((2, BR, 2 * HID + 2), lambda i: (0, i, 0)),
            pl.BlockSpec((BR, 16), lambda i: (i, 0)),
            pl.BlockSpec((8, 16), lambda i: (0, 0)),
            pl.BlockSpec((BR, HID), lambda i: (i, 0)),
        ],
        scratch_shapes=[pltpu.VMEM((1, 8), f32)],
        out_shape=[
            jax.ShapeDtypeStruct((2, N, 2 * HID + 2), f32),
            jax.ShapeDtypeStruct((N, 16), f32),
            jax.ShapeDtypeStruct((8, 16), f32),
            jax.ShapeDtypeStruct((N, HID), f32),
        ],
    )(x, w1t, acat1, wint, b_in)


@functools.lru_cache(maxsize=None)
def _make_sc_pass(n_cols, n_heads, n_rows_aa):
    cph = n_cols // n_heads
    nct = n_cols + n_heads
    mesh = plsc.VectorSubcoreMesh(core_axis_name="c", subcore_axis_name="s")

    def body(h_hbm, gsrc_hbm, dst_hbm, sc_hbm, m_hbm, zc_hbm,
             acc_out,
             acc, m_v, gsrcb_v, dstb_v,
             rows0, rows1, rows2, asg0, asg1, asg2, adg0, adg1, adg2,
             sd0, sd1, sd2, wlin_v,
             gsem0, gsem1, gsem2, ssem0, ssem1, ssem2):
        rows = (rows0, rows1, rows2)
        asg = (asg0, asg1, asg2)
        adg = (adg0, adg1, adg2)
        sdst = (sd0, sd1, sd2)
        gsem = (gsem0, gsem1, gsem2)
        ssem = (ssem0, ssem1, ssem2)
        c = lax.axis_index("c")
        s = lax.axis_index("s")
        ebase = s * EPT
        lane = jax.lax.iota(i32, 16)

        pltpu.sync_copy(m_hbm, m_v)
        pltpu.sync_copy(zc_hbm.at[pl.ds(s * RPS, RPS)],
                        acc.at[pl.ds(s * RPS, RPS)])
        plsc.subcore_barrier()
        if n_rows_aa == 8:
            mhs = [plsc.load_gather(m_v, [jnp.full((16,), 16, i32) * (2 * c + hl)])
                   for hl in range(n_heads)]
        else:
            mhs = [plsc.load_gather(m_v, [jnp.zeros((16,), i32)])]

        def refill(blk):
            pltpu.sync_copy(
                gsrc_hbm.at[pl.ds(c * E2 + ebase + blk * IB, IB)], gsrcb_v)
            pltpu.sync_copy(dst_hbm.at[pl.ds(ebase + blk * IB, IB)], dstb_v)

        def issue_gathers(local, b):
            off = local * CH
            pltpu.async_copy(h_hbm.at[gsrcb_v.at[pl.ds(off, CH)]],
                             rows[b], gsem[b])
            pltpu.async_copy(sc_hbm.at[gsrcb_v.at[pl.ds(off, CH)]],
                             asg[b], gsem[b])
            pltpu.async_copy(sc_hbm.at[dstb_v.at[pl.ds(off, CH)]],
                             adg[b], gsem[b])

        def wait_gathers(b):
            z = pl.ds(0, CH)
            pltpu.make_async_copy(h_hbm.at[gsrcb_v.at[z]], rows[b],
                                  gsem[b]).wait()
            pltpu.make_async_copy(sc_hbm.at[gsrcb_v.at[z]], asg[b],
                                  gsem[b]).wait()
            pltpu.make_async_copy(sc_hbm.at[dstb_v.at[z]], adg[b],
                                  gsem[b]).wait()

        def wait_scatter(b):
            pltpu.make_async_copy(rows[b], acc.at[sdst[b]], ssem[b]).wait()

        def copy_sdst(local, b):
            off = local * CH
            for j in range(CH // 16):
                sdst[b][pl.ds(j * 16, 16)] = dstb_v[pl.ds(off + j * 16, 16)]

        def compute_and_scatter(b):
            for j in range(CH // 16):
                lanes = lane + j * 16
                for hl in range(n_heads):
                    if n_rows_aa == 8:
                        scol = 2 * c + hl
                        dcol = 4 + 2 * c + hl
                    else:
                        scol = 0
                        dcol = 1
                    mh = mhs[hl]
                    a_s = plsc.load_gather(
                        asg[b], [lanes, jnp.full((16,), scol, i32)])
                    a_d = plsc.load_gather(
                        adg[b], [lanes, jnp.full((16,), dcol, i32)])
                    e = a_s + a_d
                    e = jnp.where(e >= 0, e, NEG * e)
                    w = jnp.exp(e - mh)
                    wlin_v[pl.ds(hl * CH + j * 16, 16)] = w
                    plsc.store_scatter(
                        rows[b], [lanes, jnp.full((16,), n_cols + hl, i32)], w)

            rb = rows[b]

            def scale_one(ei, carry2):
                for hl in range(n_heads):
                    wv = plsc.load_gather(
                        wlin_v, [jnp.full((16,), hl * CH, i32) + ei])
                    for q in range(cph // 16):
                        sl = pl.ds(hl * cph + q * 16, 16)
                        rb[ei, sl] = rb[ei, sl] * wv
                return carry2

            lax.fori_loop(0, CH, scale_one, 0, unroll=2)
            pltpu.async_copy(rb, acc.at[sdst[b]], ssem[b], add=True)

        refill(0)
        issue_gathers(0, 0)
        gpb = CPB // 3

        def group(g, carry):
            lg = lax.rem(g, gpb)
            for b in range(3):
                bn = (b + 1) % 3
                copy_sdst(3 * lg + b, b)
                wait_gathers(b)
                if b < 2:
                    @pl.when(g >= 1)
                    def _():
                        wait_scatter(bn)
                    issue_gathers(3 * lg + b + 1, bn)
                else:
                    wait_scatter(bn)

                    @pl.when(g < G3 - 1)
                    def _():
                        @pl.when(lg == gpb - 1)
                        def _():
                            refill((g + 1) // gpb)
                        issue_gathers(3 * lax.rem(g + 1, gpb), bn)
                compute_and_scatter(b)
            return carry

        lax.fori_loop(0, G3, group, 0)
        wait_scatter(1)
        wait_scatter(2)
        plsc.subcore_barrier()
        pltpu.sync_copy(acc.at[pl.ds(s * RPS, RPS)],
                        acc_out.at[pl.ds(c * NP + s * RPS, RPS)])

    return functools.partial(
        pl.kernel,
        mesh=mesh,
        compiler_params=pltpu.CompilerParams(
            needs_layout_passes=False, use_tc_tiling_on_sc=False),
        out_type=[
            jax.ShapeDtypeStruct((NCORE * NP, nct), f32),
        ],
        scratch_types=[
            pltpu.VMEM_SHARED((NP, nct), f32),
            pltpu.VMEM((128,), f32),
            pltpu.VMEM((IB,), i32),
            pltpu.VMEM((IB,), i32),
            pltpu.VMEM((CH, nct), f32),
            pltpu.VMEM((CH, nct), f32),
            pltpu.VMEM((CH, nct), f32),
            pltpu.VMEM((CH, 16), f32),
            pltpu.VMEM((CH, 16), f32),
            pltpu.VMEM((CH, 16), f32),
            pltpu.VMEM((CH, 16), f32),
            pltpu.VMEM((CH, 16), f32),
            pltpu.VMEM((CH, 16), f32),
            pltpu.VMEM((CH,), i32),
            pltpu.VMEM((CH,), i32),
            pltpu.VMEM((CH,), i32),
            pltpu.VMEM((n_heads * CH,), f32),
            pltpu.SemaphoreType.DMA,
            pltpu.SemaphoreType.DMA,
            pltpu.SemaphoreType.DMA,
            pltpu.SemaphoreType.DMA,
            pltpu.SemaphoreType.DMA,
            pltpu.SemaphoreType.DMA,
        ],
    )(body)


def _stage_b(acc1, h1p, sc1, m1, w2t, b1, acat2, kmat):
    def body(acc1_ref, h1p_ref, sc1_ref, m1_ref, w2t_ref,
             b1_ref, acat2_ref, kmat_ref,
             h2f_ref, sc2_ref, m2_ref, msc):
        i = pl.program_id(0)
        C2 = 2 * HID
        a0 = acc1_ref[0]
        a1 = acc1_ref[1]
        out1 = jnp.concatenate([a0[:, :C2], a1[:, :C2]], axis=1)
        h1 = jnp.concatenate([h1p_ref[0][:, :C2], h1p_ref[1][:, :C2]], axis=1)
        den4 = jnp.concatenate([a0[:, C2:C2 + 2], a1[:, C2:C2 + 2]],
                               axis=1)
        aat = sc1_ref[:]
        m1v = m1_ref[:][:HEADS, 0]
        es = _leaky(aat[:, :HEADS] + aat[:, HEADS:2 * HEADS])
        wself = jnp.exp(es - m1v[None, :])
        K = kmat_ref[:]
        wb = jnp.dot(wself, K, preferred_element_type=f32)
        db = jnp.dot(den4 + wself, K, preferred_element_type=f32) + 1e-16
        g1 = _leaky((out1 + wb * h1) / db + b1_ref[:])
        h2 = jnp.dot(g1, w2t_ref[:], preferred_element_type=f32)
        zpad = jnp.zeros((BR, 1), f32)
        h2f_ref[0] = jnp.concatenate([h2[:, :HID // 2], zpad], axis=1)
        h2f_ref[1] = jnp.concatenate([h2[:, HID // 2:], zpad], axis=1)
        A2 = acat2_ref[:]
        av = jnp.dot(h2, A2.T, preferred_element_type=f32)
        sc2_ref[:] = jnp.concatenate([av, jnp.zeros((BR, 14), f32)], axis=1)
        bmx = jnp.max(av, axis=0, keepdims=True)

        @pl.when(i == 0)
        def _():
            msc[:] = jnp.full((1, 2), -jnp.inf, f32)

        msc[:] = jnp.maximum(msc[:], bmx)
        mx = msc[:][0]
        m2_ref[:] = jnp.concatenate(
            [jnp.broadcast_to(mx[:1] + mx[1:], (1, 16)),
             jnp.zeros((7, 16), f32)], axis=0)

    full = lambda *shape: pl.BlockSpec(shape, lambda i: (0,) * len(shape))
    return pl.pallas_call(
        body,
        grid=(NG,),
        in_specs=[
            pl.BlockSpec((2, BR, 2 * HID + 2), lambda i: (0, i, 0)),
            pl.BlockSpec((2, BR, 2 * HID + 2), lambda i: (0, i, 0)),
            pl.BlockSpec((BR, 16), lambda i: (i, 0)),
            full(8, 16),
            full(4 * HID, HID),
            full(1, 4 * HID),
            full(2, HID),
            full(HEADS, 4 * HID),
        ],
        out_specs=[
            pl.BlockSpec((2, BR, HID // 2 + 1), lambda i: (0, i, 0)),
            pl.BlockSpec((BR, 16), lambda i: (i, 0)),
            pl.BlockSpec((8, 16), lambda i: (0, 0)),
        ],
        scratch_shapes=[pltpu.VMEM((1, 2), f32)],
        out_shape=[
            jax.ShapeDtypeStruct((2, N, HID // 2 + 1), f32),
            jax.ShapeDtypeStruct((N, 16), f32),
            jax.ShapeDtypeStruct((8, 16), f32),
        ],
    )(acc1, h1p, sc1, m1, w2t, b1, acat2, kmat)


def _stage_c(acc2, h2f, sc2, m2, xw, b2, gamma, beta,
             wiht, b_ih, b_hh, wfct, b_fc):
    def body(acc2_ref, h2f_ref, sc2_ref, m2_ref, xw_ref,
             b2_ref, gamma_ref, beta_ref, wiht_ref, bih_ref, bhh_ref,
             wfct_ref, bfc_ref, out_ref):
        CC = HID // 2
        b0 = acc2_ref[0]
        b1v = acc2_ref[1]
        out2 = jnp.concatenate([b0[:, :CC], b1v[:, :CC]], axis=1)
        h2 = jnp.concatenate([h2f_ref[0][:, :CC], h2f_ref[1][:, :CC]], axis=1)
        den = b0[:, CC:CC + 1]
        at = sc2_ref[:]
        wself = jnp.exp(_leaky(at[:, :1] + at[:, 1:2]) - m2_ref[0, 0])
        g2 = (out2 + wself * h2) / (den + wself + 1e-16)
        h = _leaky(g2 + b2_ref[:])
        mu = jnp.mean(h, axis=1, keepdims=True)
        d = h - mu
        var = jnp.mean(d * d, axis=1, keepdims=True)
        h = d * jax.lax.rsqrt(var + 1e-5) * gamma_ref[:] + beta_ref[:]
        h = h + xw_ref[:]
        gi = jnp.dot(h, wiht_ref[:], preferred_element_type=f32) + bih_ref[:]
        bhh = bhh_ref[:]
        r = jax.nn.sigmoid(gi[:, :HID] + bhh[:, :HID])
        z = jax.nn.sigmoid(gi[:, HID:2 * HID] + bhh[:, HID:2 * HID])
        nc = jnp.tanh(gi[:, 2 * HID:] + r * bhh[:, 2 * HID:])
        hout = (1.0 - z) * nc
        out_ref[:] = jnp.dot(hout, wfct_ref[:],
                             preferred_element_type=f32) + bfc_ref[:]

    full = lambda *shape: pl.BlockSpec(shape, lambda i: (0,) * len(shape))
    return pl.pallas_call(
        body,
        grid=(NG,),
        in_specs=[
            pl.BlockSpec((2, BR, HID // 2 + 1), lambda i: (0, i, 0)),
            pl.BlockSpec((2, BR, HID // 2 + 1), lambda i: (0, i, 0)),
            pl.BlockSpec((BR, 16), lambda i: (i, 0)),
            full(8, 16),
            pl.BlockSpec((BR, HID), lambda i: (i, 0)),
            full(1, HID),
            full(1, HID),
            full(1, HID),
            full(HID, 3 * HID),
            full(1, 3 * HID),
            full(1, 3 * HID),
            full(HID, 8),
            full(1, 8),
        ],
        out_specs=pl.BlockSpec((BR, 8), lambda i: (i, 0)),
        out_shape=jax.ShapeDtypeStruct((N, 8), f32),
    )(acc2, h2f, sc2, m2, xw, b2, gamma, beta,
      wiht, b_ih, b_hh, wfct, b_fc)


def _sc_pass1(*args):
    return _make_sc_pass(2 * HID, 2, 8)(*args)


def _sc_pass2(*args):
    return _make_sc_pass(HID // 2, 1, 2)(*args)


@jax.jit
def kernel(x, edge_index, W_in, b_in, W1, a_src1, a_dst1, b1, W2, a_src2,
           a_dst2, b2, gamma, beta, W_ih, W_hh, b_ih, b_hh, W_fc, b_fc):
    w1t = W1.T
    W1h = W1.reshape(HEADS, HID, F_IN)
    as_rows = jnp.einsum('hcf,hc->hf', W1h, a_src1)
    ad_rows = jnp.einsum('hcf,hc->hf', W1h, a_dst1)
    acat1 = jnp.concatenate([as_rows, ad_rows], axis=0)
    wint = W_in.T
    w2t = W2.T
    acat2 = jnp.concatenate([a_src2, a_dst2], axis=0)
    kmat = jnp.kron(jnp.eye(HEADS, dtype=f32), jnp.ones((1, HID), f32))
    wiht = W_ih.T
    wfct = jnp.concatenate(
        [W_fc.T, jnp.zeros((HID, 5), f32)], axis=1)
    bfc = jnp.concatenate([b_fc, jnp.zeros((5,), f32)])[None, :]

    pade = E2 - E
    src_p = jnp.concatenate([edge_index[0], jnp.zeros((pade,), i32)])
    dst_p = jnp.concatenate([edge_index[1], jnp.full((pade,), N, i32)])
    gsrc_all = jnp.concatenate([src_p, src_p + N])
    zc1 = jnp.zeros((NP, 2 * HID + 2), f32)
    zc2 = jnp.zeros((NP, HID // 2 + 1), f32)

    h1p, sc1, m1, xw = _stage_a(x, w1t, acat1, wint, b_in[None, :])
    sc1x = jnp.concatenate([sc1, sc1], axis=0)

    acc1 = _sc_pass1(h1p.reshape(NCORE * N, 2 * HID + 2), gsrc_all, dst_p,
                     sc1x, m1.reshape(-1), zc1)
    acc1 = acc1[0] if isinstance(acc1, (list, tuple)) else acc1

    acc1c = jnp.stack([acc1[:N], acc1[NP:NP + N]])
    h2f, sc2, m2 = _stage_b(
        acc1c, h1p, sc1, m1, w2t, b1[None, :], acat2, kmat)
    sc2x = jnp.concatenate([sc2, sc2], axis=0)

    acc2 = _sc_pass2(h2f.reshape(NCORE * N, HID // 2 + 1), gsrc_all, dst_p,
                     sc2x, m2.reshape(-1), zc2)
    acc2 = acc2[0] if isinstance(acc2, (list, tuple)) else acc2

    acc2c = jnp.stack([acc2[:N], acc2[NP:NP + N]])
    out = _stage_c(acc2c, h2f, sc2, m2, xw,
                   b2[None, :], gamma[None, :], beta[None, :],
                   wiht, b_ih[None, :], b_hh[None, :], wfct, bfc)
    return out[:, :3]
